# baseline (speedup 1.0000x reference)
import jax
import jax.numpy as jnp
from jax.experimental import pallas as pl


def _layer(u_emb, i_emb, edge_index, weights):
    src_u = edge_index[0]
    dst_i = edge_index[1]
    msg_to_u = weights[:, None] * jnp.take(i_emb, dst_i, axis=0)
    u_out = jax.ops.segment_sum(msg_to_u, src_u, num_segments=u_emb.shape[0])
    msg_to_i = weights[:, None] * jnp.take(u_emb, src_u, axis=0)
    i_out = jax.ops.segment_sum(msg_to_i, dst_i, num_segments=i_emb.shape[0])
    return u_out, i_out


def _dot_kernel(u_ref, i_ref, o_ref):
    o_ref[:] = jnp.sum(u_ref[:] * i_ref[:], axis=-1)


def kernel(user_indices, item_indices, edge_index_t0, weights_t0,
           edge_index_t1, weights_t1, user_table, item_table, type_weights):
    tw = jax.nn.softmax(type_weights, axis=0)
    u_emb, i_emb = user_table, item_table
    all_u, all_i = [u_emb], [i_emb]
    for _ in range(2):
        u_t0, i_t0 = _layer(u_emb, i_emb, edge_index_t0, weights_t0)
        u_t1, i_t1 = _layer(u_emb, i_emb, edge_index_t1, weights_t1)
        u_emb = tw[0] * u_t0 + tw[1] * u_t1
        i_emb = tw[0] * i_t0 + tw[1] * i_t1
        all_u.append(u_emb)
        all_i.append(i_emb)
    final_u = jnp.stack(all_u, axis=1).mean(axis=1)
    final_i = jnp.stack(all_i, axis=1).mean(axis=1)
    u_final = jnp.take(final_u, user_indices, axis=0)
    i_final = jnp.take(final_i, item_indices, axis=0)
    return pl.pallas_call(
        _dot_kernel,
        out_shape=jax.ShapeDtypeStruct((u_final.shape[0],), jnp.float32),
    )(u_final, i_final)


# speedup vs baseline: 1.5711x; 1.5711x over previous
import functools

import jax
import jax.numpy as jnp
import numpy as np
from jax import lax
from jax.experimental import pallas as pl
from jax.experimental.pallas import tpu as pltpu
from jax.experimental.pallas import tpu_sc as plsc

N_NODES = 50000
D = 64
E_RAW = 800000
HALF = 25088
NPAD = 2 * HALF
ACC_ROWS = HALF + 16
E_PAD = 819200
SUP = 1024
SUPS_PER_TILE = E_PAD // (16 * SUP)
CHUNK = 128
ZROWS = ACC_ROWS // 16
WROWS = HALF // 16

_mesh = plsc.VectorSubcoreMesh(core_axis_name="c", subcore_axis_name="s")


def _softmax2(tw_v):
    t = tw_v[...]
    mx = jnp.full((16,), jnp.maximum(t[0], t[1]), jnp.float32)
    e = jnp.exp(t - mx)
    e0 = jnp.full((16,), e[0], jnp.float32)
    e1 = jnp.full((16,), e[1], jnp.float32)
    ssum = e0 + e1
    return e0 / ssum, e1 / ssum


_GDN = lax.GatherDimensionNumbers(
    offset_dims=(), collapsed_slice_dims=(0,), start_index_map=(0,))


def _permute(v, idx):
    return lax.gather(v, idx[:, None], _GDN, (1,),
                      mode=lax.GatherScatterMode.PROMISE_IN_BOUNDS)


def _lane_sum(v):
    lanes = lax.iota(jnp.int32, 16)
    for k in (8, 4, 2, 1):
        v = v + _permute(v, lanes ^ k)
    return v


def _splat0(v):
    return _permute(v, jnp.zeros((16,), jnp.int32))


def _phase(c, s, streams, table_ref, out_ref,
           tgt_s, oth_s, w_s, cw_v, coth_v, ctl_v, sidx_v, rows_v, acc,
           stg0, stg1, gsem, ssem):
    lo = c * HALF
    stgsem = (stg0, stg1)
    def zrow(r, carry):
        for d in range(4):
            rows_v[0, r, pl.ds(d * 16, 16)] = jnp.zeros((16,), jnp.float32)
        return carry
    lax.fori_loop(0, CHUNK, zrow, 0)
    zoff = s * ZROWS
    nfull = ZROWS // CHUNK
    for kk in range(nfull):
        pltpu.sync_copy(rows_v.at[0],
                        acc.at[pl.ds(zoff + kk * CHUNK, CHUNK)])
    pltpu.sync_copy(rows_v.at[0, pl.ds(0, ZROWS - nfull * CHUNK)],
                    acc.at[pl.ds(zoff + nfull * CHUNK, ZROWS - nfull * CHUNK)])
    plsc.subcore_barrier()

    base = s * (SUPS_PER_TILE * SUP)
    gc = jnp.int32(0)

    def czero(j, carry):
        coth_v[pl.ds(j * 16, 16)] = jnp.zeros((16,), jnp.int32)
        return carry
    lax.fori_loop(0, (SUP + 16) // 16, czero, 0)

    for tgt_hbm, oth_hbm, wgt_hbm, tws in streams:
        def stage(bb, sup):
            off = base + sup * SUP
            pltpu.async_copy(tgt_hbm.at[pl.ds(off, SUP)],
                             tgt_s.at[bb, pl.ds(0, SUP)], stgsem[bb])
            pltpu.async_copy(oth_hbm.at[pl.ds(off, SUP)],
                             oth_s.at[bb, pl.ds(0, SUP)], stgsem[bb])
            pltpu.async_copy(wgt_hbm.at[pl.ds(off, SUP)],
                             w_s.at[bb, pl.ds(0, SUP)], stgsem[bb])

        def super_body(bb, sup, i, gc):
            for hb, st in ((tgt_hbm, tgt_s), (oth_hbm, oth_s), (wgt_hbm, w_s)):
                pltpu.make_async_copy(hb.at[pl.ds(0, SUP)],
                                      st.at[bb, pl.ds(0, SUP)],
                                      stgsem[bb]).wait()

            def pre(j, carry):
                cw_v[pl.ds(j * 16, 16)] = jnp.zeros((16,), jnp.float32)
                ctl_v[pl.ds(j * 16, 16)] = jnp.full((16,), HALF, jnp.int32)
                return carry
            lax.fori_loop(0, (SUP + 16) // 16, pre, 0)

            def comp(j, cur):
                sl = pl.ds(j * 16, 16)
                t = tgt_s[bb, sl]
                m = (t >= lo) & (t < lo + HALF)
                wv = w_s[bb, sl] * tws
                plsc.store_compressed(cw_v.at[pl.ds(cur, 16)], wv, mask=m)
                plsc.store_compressed(coth_v.at[pl.ds(cur, 16)],
                                      oth_s[bb, sl], mask=m)
                plsc.store_compressed(ctl_v.at[pl.ds(cur, 16)], t - lo, mask=m)
                pc = plsc.all_reduce_population_count(m)
                return cur + pc[0]
            k = lax.fori_loop(0, SUP // 16, comp, jnp.int32(0))

            nck = (k + CHUNK - 1) // CHUNK

            def cchunk(cidx, gc):
                rb = gc & 1

                @pl.when(gc >= 2)
                def _drain():
                    pltpu.make_async_copy(out_ref.at[pl.ds(0, CHUNK)],
                                          rows_v.at[rb], ssem.at[rb]).wait()
                h = CHUNK // 2
                cp = pltpu.async_copy(
                    table_ref.at[coth_v.at[pl.ds(cidx * CHUNK, h)]],
                    rows_v.at[rb, pl.ds(0, h)], gsem)
                cp2 = pltpu.async_copy(
                    table_ref.at[coth_v.at[pl.ds(cidx * CHUNK + h, h)]],
                    rows_v.at[rb, pl.ds(h, h)], stg0)

                def sloop(j, carry):
                    sidx_v[rb, pl.ds(j * 16, 16)] = (
                        ctl_v[pl.ds(cidx * CHUNK + j * 16, 16)])
                    return carry
                lax.fori_loop(0, CHUNK // 16, sloop, 0)
                cp.wait()
                cp2.wait()

                def rm(r, carry):
                    ws = _splat0(cw_v[pl.ds(cidx * CHUNK + r, 16)])
                    for d in range(4):
                        sl = pl.ds(d * 16, 16)
                        rows_v[rb, r, sl] = rows_v[rb, r, sl] * ws
                    return carry
                lax.fori_loop(0, CHUNK, rm, 0)

                pltpu.async_copy(rows_v.at[rb], acc.at[sidx_v.at[rb]],
                                 ssem.at[rb], add=True)
                return gc + 1
            gc = lax.fori_loop(0, nck, cchunk, gc)

            @pl.when(i < SUPS_PER_TILE // 2 - 1)
            def _prefetch():
                stage(bb, sup + 2)
            return gc

        stage(0, 0)
        stage(1, 1)

        def pair(i, gc):
            gc = super_body(0, 2 * i, i, gc)
            gc = super_body(1, 2 * i + 1, i, gc)
            return gc
        gc = lax.fori_loop(0, SUPS_PER_TILE // 2, pair, gc)

    @pl.when(gc >= 2)
    def _d0():
        pltpu.make_async_copy(out_ref.at[pl.ds(0, CHUNK)],
                              rows_v.at[gc & 1], ssem.at[gc & 1]).wait()

    @pl.when(gc >= 1)
    def _d1():
        pltpu.make_async_copy(out_ref.at[pl.ds(0, CHUNK)],
                              rows_v.at[(gc - 1) & 1],
                              ssem.at[(gc - 1) & 1]).wait()

    plsc.subcore_barrier()
    woff = s * WROWS
    pltpu.sync_copy(acc.at[pl.ds(woff, WROWS)],
                    out_ref.at[pl.ds(lo + woff, WROWS)])
    plsc.subcore_barrier()


@functools.partial(
    pl.kernel,
    out_type=[jax.ShapeDtypeStruct((NPAD, D), jnp.float32),
              jax.ShapeDtypeStruct((NPAD, D), jnp.float32)],
    mesh=_mesh,
    compiler_params=pltpu.CompilerParams(use_tc_tiling_on_sc=False, needs_layout_passes=False),
    scratch_types=[
        pltpu.VMEM((2, SUP), jnp.int32),
        pltpu.VMEM((2, SUP), jnp.int32),
        pltpu.VMEM((2, SUP), jnp.float32),
        pltpu.VMEM((SUP + 16,), jnp.float32),
        pltpu.VMEM((SUP + 16,), jnp.int32),
        pltpu.VMEM((SUP + 16,), jnp.int32),
        pltpu.VMEM((2, CHUNK), jnp.int32),
        pltpu.VMEM((2, CHUNK, D), jnp.float32),
        pltpu.VMEM((16,), jnp.float32),
        pltpu.VMEM_SHARED((ACC_ROWS, D), jnp.float32),
        pltpu.SemaphoreType.DMA,
        pltpu.SemaphoreType.DMA,
        pltpu.SemaphoreType.DMA,
        pltpu.SemaphoreType.DMA((2,)),
    ],
)
def _layer_kernel(u_in, i_in, s0, d0, w0, s1, d1, w1, twp,
                  u_out, i_out,
                  tgt_s, oth_s, w_s, cw_v, coth_v, ctl_v, sidx_v, rows_v,
                  tw_v, acc, stg0, stg1, gsem, ssem):
    c = lax.axis_index("c")
    s = lax.axis_index("s")
    pltpu.sync_copy(twp, tw_v)
    tw0, tw1 = _softmax2(tw_v)
    _phase(c, s, [(s0, d0, w0, tw0), (s1, d1, w1, tw1)], i_in, u_out,
           tgt_s, oth_s, w_s, cw_v, coth_v, ctl_v, sidx_v, rows_v, acc,
           stg0, stg1, gsem, ssem)
    _phase(c, s, [(d0, s0, w0, tw0), (d1, s1, w1, tw1)], u_in, i_out,
           tgt_s, oth_s, w_s, cw_v, coth_v, ctl_v, sidx_v, rows_v, acc,
           stg0, stg1, gsem, ssem)


@functools.partial(
    pl.kernel,
    out_type=jax.ShapeDtypeStruct((4096, ), jnp.float32),
    mesh=_mesh,
    compiler_params=pltpu.CompilerParams(use_tc_tiling_on_sc=False, needs_layout_passes=False),
    scratch_types=[
        pltpu.VMEM((128,), jnp.int32),
        pltpu.VMEM((128,), jnp.int32),
        pltpu.VMEM((128, D), jnp.float32),
        pltpu.VMEM((128, D), jnp.float32),
        pltpu.VMEM((128, D), jnp.float32),
        pltpu.VMEM((128, D), jnp.float32),
        pltpu.VMEM((128, D), jnp.float32),
        pltpu.VMEM((128, D), jnp.float32),
        pltpu.VMEM((128,), jnp.float32),
        pltpu.SemaphoreType.DMA,
    ],
)
def _final_kernel(u0, u1, u2, i0, i1, i2, ui_hbm, ii_hbm, out_hbm,
                  uidx_v, iidx_v, u0r, u1r, u2r, i0r, i1r, i2r, out_v, sem):
    wid = lax.axis_index("s") * 2 + lax.axis_index("c")
    base = wid * 128
    pltpu.sync_copy(ui_hbm.at[pl.ds(base, 128)], uidx_v)
    pltpu.sync_copy(ii_hbm.at[pl.ds(base, 128)], iidx_v)
    cps = [pltpu.async_copy(t.at[ix], dst, sem)
           for (t, ix, dst) in [(u0, uidx_v, u0r), (u1, uidx_v, u1r),
                                (u2, uidx_v, u2r), (i0, iidx_v, i0r),
                                (i1, iidx_v, i1r), (i2, iidx_v, i2r)]]
    for cp in cps:
        cp.wait()
    lane0 = lax.iota(jnp.int32, 16) == 0

    def row(r, carry):
        p = jnp.zeros((16,), jnp.float32)
        for d in range(4):
            sl = pl.ds(d * 16, 16)
            su = u0r[r, sl] + u1r[r, sl] + u2r[r, sl]
            si = i0r[r, sl] + i1r[r, sl] + i2r[r, sl]
            p = p + su * si
        tot = _lane_sum(p) * jnp.float32(1.0 / 9.0)
        plsc.store_scatter(out_v, [jnp.full((16,), r, jnp.int32)],
                           tot, mask=lane0)
        return carry
    lax.fori_loop(0, 128, row, 0)
    pltpu.sync_copy(out_v, out_hbm.at[pl.ds(base, 128)])


def kernel(user_indices, item_indices, edge_index_t0, weights_t0,
           edge_index_t1, weights_t1, user_table, item_table, type_weights):
    epad = E_PAD - E_RAW
    s0 = jnp.pad(edge_index_t0[0], (0, epad))
    d0 = jnp.pad(edge_index_t0[1], (0, epad))
    w0 = jnp.pad(weights_t0, (0, epad))
    s1 = jnp.pad(edge_index_t1[0], (0, epad))
    d1 = jnp.pad(edge_index_t1[1], (0, epad))
    w1 = jnp.pad(weights_t1, (0, epad))
    u0p = jnp.pad(user_table, ((0, NPAD - N_NODES), (0, 0)))
    i0p = jnp.pad(item_table, ((0, NPAD - N_NODES), (0, 0)))
    twp = jnp.pad(type_weights, (0, 14), constant_values=-1e30)

    u1p, i1p = _layer_kernel(u0p, i0p, s0, d0, w0, s1, d1, w1, twp)
    u2p, i2p = _layer_kernel(u1p, i1p, s0, d0, w0, s1, d1, w1, twp)
    return _final_kernel(u0p, u1p, u2p, i0p, i1p, i2p,
                         user_indices, item_indices)


# speedup vs baseline: 2.7423x; 1.7454x over previous
import functools

import jax
import jax.numpy as jnp
import numpy as np
from jax import lax
from jax.experimental import pallas as pl
from jax.experimental.pallas import tpu as pltpu
from jax.experimental.pallas import tpu_sc as plsc

N_NODES = 50000
D = 64
E_RAW = 800000
HALF = 25088
NPAD = 2 * HALF
ACC_ROWS = HALF + 16
E_PAD = 819200
SUP = 1024
SUPS_PER_TILE = E_PAD // (16 * SUP)
CHUNK = 128
ZROWS = ACC_ROWS // 16
WROWS = HALF // 16

_mesh = plsc.VectorSubcoreMesh(core_axis_name="c", subcore_axis_name="s")


def _softmax2(tw_v):
    t = tw_v[...]
    mx = jnp.full((16,), jnp.maximum(t[0], t[1]), jnp.float32)
    e = jnp.exp(t - mx)
    e0 = jnp.full((16,), e[0], jnp.float32)
    e1 = jnp.full((16,), e[1], jnp.float32)
    ssum = e0 + e1
    return e0 / ssum, e1 / ssum


_GDN = lax.GatherDimensionNumbers(
    offset_dims=(), collapsed_slice_dims=(0,), start_index_map=(0,))


def _permute(v, idx):
    return lax.gather(v, idx[:, None], _GDN, (1,),
                      mode=lax.GatherScatterMode.PROMISE_IN_BOUNDS)


def _lane_sum(v):
    lanes = lax.iota(jnp.int32, 16)
    for k in (8, 4, 2, 1):
        v = v + _permute(v, lanes ^ k)
    return v


def _splat0(v):
    return _permute(v, jnp.zeros((16,), jnp.int32))


def _phase(c, s, streams, table_ref, out_ref,
           tgt_s, oth_s, w_s, cw_v, coth_v, ctl_v, sidx_v, rows_v, acc,
           stg0, stg1, gsem, ssem):
    lo = c * HALF
    stgsem = (stg0, stg1)
    def zrow(r, carry):
        for d in range(4):
            rows_v[0, r, pl.ds(d * 16, 16)] = jnp.zeros((16,), jnp.float32)
        return carry
    lax.fori_loop(0, CHUNK, zrow, 0)
    zoff = s * ZROWS
    nfull = ZROWS // CHUNK
    for kk in range(nfull):
        pltpu.sync_copy(rows_v.at[0],
                        acc.at[pl.ds(zoff + kk * CHUNK, CHUNK)])
    pltpu.sync_copy(rows_v.at[0, pl.ds(0, ZROWS - nfull * CHUNK)],
                    acc.at[pl.ds(zoff + nfull * CHUNK, ZROWS - nfull * CHUNK)])
    plsc.subcore_barrier()

    base = s * (SUPS_PER_TILE * SUP)
    gc = jnp.int32(0)

    def czero(j, carry):
        coth_v[pl.ds(j * 16, 16)] = jnp.zeros((16,), jnp.int32)
        return carry
    lax.fori_loop(0, (SUP + 16) // 16, czero, 0)

    for tgt_hbm, oth_hbm, wgt_hbm, tws in streams:
        def stage(bb, sup):
            off = base + sup * SUP
            pltpu.async_copy(tgt_hbm.at[pl.ds(off, SUP)],
                             tgt_s.at[bb, pl.ds(0, SUP)], stgsem[bb])
            pltpu.async_copy(oth_hbm.at[pl.ds(off, SUP)],
                             oth_s.at[bb, pl.ds(0, SUP)], stgsem[bb])
            pltpu.async_copy(wgt_hbm.at[pl.ds(off, SUP)],
                             w_s.at[bb, pl.ds(0, SUP)], stgsem[bb])

        def super_body(bb, sup, i, gc):
            for hb, st in ((tgt_hbm, tgt_s), (oth_hbm, oth_s), (wgt_hbm, w_s)):
                pltpu.make_async_copy(hb.at[pl.ds(0, SUP)],
                                      st.at[bb, pl.ds(0, SUP)],
                                      stgsem[bb]).wait()

            def pre(j, carry):
                cw_v[pl.ds(j * 16, 16)] = jnp.zeros((16,), jnp.float32)
                ctl_v[pl.ds(j * 16, 16)] = jnp.full((16,), HALF, jnp.int32)
                return carry
            lax.fori_loop(0, (SUP + 16) // 16, pre, 0)

            def comp(j, cur):
                sl = pl.ds(j * 16, 16)
                t = tgt_s[bb, sl]
                m = (t >= lo) & (t < lo + HALF)
                wv = w_s[bb, sl] * tws
                plsc.store_compressed(cw_v.at[pl.ds(cur, 16)], wv, mask=m)
                plsc.store_compressed(coth_v.at[pl.ds(cur, 16)],
                                      oth_s[bb, sl], mask=m)
                plsc.store_compressed(ctl_v.at[pl.ds(cur, 16)], t - lo, mask=m)
                pc = plsc.all_reduce_population_count(m)
                return cur + pc[0]
            k = lax.fori_loop(0, SUP // 16, comp, jnp.int32(0))

            nck = (k + CHUNK - 1) // CHUNK

            def cchunk(cidx, gc):
                rb = gc & 1

                @pl.when(gc >= 2)
                def _drain():
                    pltpu.make_async_copy(out_ref.at[pl.ds(0, CHUNK)],
                                          rows_v.at[rb], ssem.at[rb]).wait()
                cp = pltpu.async_copy(
                    table_ref.at[coth_v.at[pl.ds(cidx * CHUNK, CHUNK)]],
                    rows_v.at[rb], gsem)

                def sloop(j, carry):
                    sidx_v[rb, pl.ds(j * 16, 16)] = (
                        ctl_v[pl.ds(cidx * CHUNK + j * 16, 16)])
                    return carry
                lax.fori_loop(0, CHUNK // 16, sloop, 0)
                cp.wait()

                def rm(r, carry):
                    ws = _splat0(cw_v[pl.ds(cidx * CHUNK + r, 16)])
                    for d in range(4):
                        sl = pl.ds(d * 16, 16)
                        rows_v[rb, r, sl] = rows_v[rb, r, sl] * ws
                    return carry
                lax.fori_loop(0, CHUNK, rm, 0)

                pltpu.async_copy(rows_v.at[rb], acc.at[sidx_v.at[rb]],
                                 ssem.at[rb], add=True)
                return gc + 1
            gc = lax.fori_loop(0, nck, cchunk, gc)

            @pl.when(i < SUPS_PER_TILE // 2 - 1)
            def _prefetch():
                stage(bb, sup + 2)
            return gc

        stage(0, 0)
        stage(1, 1)

        def pair(i, gc):
            gc = super_body(0, 2 * i, i, gc)
            gc = super_body(1, 2 * i + 1, i, gc)
            return gc
        gc = lax.fori_loop(0, SUPS_PER_TILE // 2, pair, gc)

    @pl.when(gc >= 2)
    def _d0():
        pltpu.make_async_copy(out_ref.at[pl.ds(0, CHUNK)],
                              rows_v.at[gc & 1], ssem.at[gc & 1]).wait()

    @pl.when(gc >= 1)
    def _d1():
        pltpu.make_async_copy(out_ref.at[pl.ds(0, CHUNK)],
                              rows_v.at[(gc - 1) & 1],
                              ssem.at[(gc - 1) & 1]).wait()

    plsc.subcore_barrier()
    woff = s * WROWS
    pltpu.sync_copy(acc.at[pl.ds(woff, WROWS)],
                    out_ref.at[pl.ds(lo + woff, WROWS)])
    plsc.subcore_barrier()


@functools.partial(
    pl.kernel,
    out_type=[jax.ShapeDtypeStruct((NPAD, D), jnp.float32),
              jax.ShapeDtypeStruct((NPAD, D), jnp.float32)],
    mesh=_mesh,
    compiler_params=pltpu.CompilerParams(use_tc_tiling_on_sc=False, needs_layout_passes=False),
    scratch_types=[
        pltpu.VMEM((2, SUP), jnp.int32),
        pltpu.VMEM((2, SUP), jnp.int32),
        pltpu.VMEM((2, SUP), jnp.float32),
        pltpu.VMEM((SUP + 16,), jnp.float32),
        pltpu.VMEM((SUP + 16,), jnp.int32),
        pltpu.VMEM((SUP + 16,), jnp.int32),
        pltpu.VMEM((2, CHUNK), jnp.int32),
        pltpu.VMEM((2, CHUNK, D), jnp.float32),
        pltpu.VMEM((16,), jnp.float32),
        pltpu.VMEM_SHARED((ACC_ROWS, D), jnp.float32),
        pltpu.SemaphoreType.DMA,
        pltpu.SemaphoreType.DMA,
        pltpu.SemaphoreType.DMA,
        pltpu.SemaphoreType.DMA((2,)),
    ],
)
def _layer_kernel(u_in, i_in, s0, d0, w0, s1, d1, w1, twp,
                  u_out, i_out,
                  tgt_s, oth_s, w_s, cw_v, coth_v, ctl_v, sidx_v, rows_v,
                  tw_v, acc, stg0, stg1, gsem, ssem):
    c = lax.axis_index("c")
    s = lax.axis_index("s")
    pltpu.sync_copy(twp, tw_v)
    tw0, tw1 = _softmax2(tw_v)
    _phase(c, s, [(s0, d0, w0, tw0), (s1, d1, w1, tw1)], i_in, u_out,
           tgt_s, oth_s, w_s, cw_v, coth_v, ctl_v, sidx_v, rows_v, acc,
           stg0, stg1, gsem, ssem)
    _phase(c, s, [(d0, s0, w0, tw0), (d1, s1, w1, tw1)], u_in, i_out,
           tgt_s, oth_s, w_s, cw_v, coth_v, ctl_v, sidx_v, rows_v, acc,
           stg0, stg1, gsem, ssem)


BATCH = 4096
BSLOTS = BATCH + 128
BZ = BSLOTS // 16


@functools.partial(
    pl.kernel,
    out_type=[jax.ShapeDtypeStruct((BATCH, D), jnp.float32),
              jax.ShapeDtypeStruct((BATCH, D), jnp.float32)],
    mesh=_mesh,
    compiler_params=pltpu.CompilerParams(use_tc_tiling_on_sc=False, needs_layout_passes=False),
    scratch_types=[
        pltpu.VMEM((NPAD,), jnp.int32),
        pltpu.VMEM((2, SUP), jnp.int32),
        pltpu.VMEM((2, SUP), jnp.int32),
        pltpu.VMEM((2, SUP), jnp.float32),
        pltpu.VMEM((SUP + 16,), jnp.float32),
        pltpu.VMEM((SUP + 16,), jnp.int32),
        pltpu.VMEM((SUP + 16,), jnp.int32),
        pltpu.VMEM((2, CHUNK), jnp.int32),
        pltpu.VMEM((2, CHUNK, D), jnp.float32),
        pltpu.VMEM((CHUNK,), jnp.int32),
        pltpu.VMEM((16,), jnp.float32),
        pltpu.VMEM_SHARED((BSLOTS, D), jnp.float32),
        pltpu.SemaphoreType.DMA,
        pltpu.SemaphoreType.DMA,
        pltpu.SemaphoreType.DMA,
        pltpu.SemaphoreType.DMA((2,)),
    ],
)
def _tail_kernel(u0, i0, u1, i1, s0, d0, w0, s1, d1, w1, twp, ui_hbm, ii_hbm,
                 su_out, si_out,
                 inv_v, tgt_s, oth_s, w_s, cw_v, coth_v, ctl_v, sidx_v,
                 rows_v, bidx_v, tw_v, acc, stg0, stg1, gsem, ssem):
    c = lax.axis_index("c")
    s = lax.axis_index("s")
    pltpu.sync_copy(twp, tw_v)
    tw0, tw1 = _softmax2(tw_v)

    def side(bat_hbm, t0_hbm, t1_hbm, src_tbl, out_hbm, streams):
        lo = 0
        stgsem = (stg0, stg1)
        def minv(j, carry):
            inv_v[pl.ds(j * 16, 16)] = jnp.full((16,), -1, jnp.int32)
            return carry
        lax.fori_loop(0, NPAD // 16, minv, 0)
        lane0 = lax.iota(jnp.int32, 16) == 0
        for q in range(BATCH // SUP):
            pltpu.sync_copy(bat_hbm.at[pl.ds(q * SUP, SUP)],
                            tgt_s.at[0, pl.ds(0, SUP)])

            def sc1(b, carry):
                uib = plsc.load_gather(tgt_s.at[0], [jnp.full((16,), b,
                                                              jnp.int32)])
                plsc.store_scatter(inv_v, [uib],
                                   jnp.full((16,), q * SUP, jnp.int32) + b,
                                   mask=lane0)
                return carry
            lax.fori_loop(0, SUP, sc1, 0)

        def zrow(r, carry):
            for d in range(4):
                rows_v[0, r, pl.ds(d * 16, 16)] = jnp.zeros((16,),
                                                            jnp.float32)
            return carry
        lax.fori_loop(0, CHUNK, zrow, 0)
        zoff = s * BZ
        for kk in range(BZ // CHUNK):
            pltpu.sync_copy(rows_v.at[0],
                            acc.at[pl.ds(zoff + kk * CHUNK, CHUNK)])
        pltpu.sync_copy(rows_v.at[0, pl.ds(0, BZ % CHUNK)],
                        acc.at[pl.ds(zoff + (BZ // CHUNK) * CHUNK,
                                     BZ % CHUNK)])
        plsc.subcore_barrier()

        base = s * (SUPS_PER_TILE * SUP)
        gc = jnp.int32(0)

        def czero(j, carry):
            coth_v[pl.ds(j * 16, 16)] = jnp.zeros((16,), jnp.int32)
            return carry
        lax.fori_loop(0, (SUP + 16) // 16, czero, 0)

        for tgt_hbm, oth_hbm, wgt_hbm, tws in streams:
            def stage(bb, sup):
                off = base + sup * SUP
                pltpu.async_copy(tgt_hbm.at[pl.ds(off, SUP)],
                                 tgt_s.at[bb, pl.ds(0, SUP)], stgsem[bb])
                pltpu.async_copy(oth_hbm.at[pl.ds(off, SUP)],
                                 oth_s.at[bb, pl.ds(0, SUP)], stgsem[bb])
                pltpu.async_copy(wgt_hbm.at[pl.ds(off, SUP)],
                                 w_s.at[bb, pl.ds(0, SUP)], stgsem[bb])

            def super_body(bb, sup, i, gc):
                for hb, st in ((tgt_hbm, tgt_s), (oth_hbm, oth_s),
                               (wgt_hbm, w_s)):
                    pltpu.make_async_copy(hb.at[pl.ds(0, SUP)],
                                          st.at[bb, pl.ds(0, SUP)],
                                          stgsem[bb]).wait()

                def pre(j, carry):
                    cw_v[pl.ds(j * 16, 16)] = jnp.zeros((16,), jnp.float32)
                    ctl_v[pl.ds(j * 16, 16)] = jnp.full((16,), BATCH,
                                                        jnp.int32)
                    return carry
                lax.fori_loop(0, (SUP + 16) // 16, pre, 0)

                def comp(j, cur):
                    sl = pl.ds(j * 16, 16)
                    t = tgt_s[bb, sl]
                    p = plsc.load_gather(inv_v, [t])
                    m = p >= 0
                    wv = w_s[bb, sl] * tws
                    plsc.store_compressed(cw_v.at[pl.ds(cur, 16)], wv,
                                          mask=m)
                    plsc.store_compressed(coth_v.at[pl.ds(cur, 16)],
                                          oth_s[bb, sl], mask=m)
                    plsc.store_compressed(ctl_v.at[pl.ds(cur, 16)], p,
                                          mask=m)
                    pc = plsc.all_reduce_population_count(m)
                    return cur + pc[0]
                k = lax.fori_loop(0, SUP // 16, comp, jnp.int32(0))
                nck = (k + CHUNK - 1) // CHUNK

                def cchunk(cidx, gc):
                    rb = gc & 1

                    @pl.when(gc >= 2)
                    def _drain():
                        pltpu.make_async_copy(out_hbm.at[pl.ds(0, CHUNK)],
                                              rows_v.at[rb],
                                              ssem.at[rb]).wait()
                    cp = pltpu.async_copy(
                        src_tbl.at[coth_v.at[pl.ds(cidx * CHUNK, CHUNK)]],
                        rows_v.at[rb], gsem)

                    def sloop(j, carry):
                        sidx_v[rb, pl.ds(j * 16, 16)] = (
                            ctl_v[pl.ds(cidx * CHUNK + j * 16, 16)])
                        return carry
                    lax.fori_loop(0, CHUNK // 16, sloop, 0)
                    cp.wait()

                    def rm(r, carry):
                        ws = _splat0(cw_v[pl.ds(cidx * CHUNK + r, 16)])
                        for d in range(4):
                            sl = pl.ds(d * 16, 16)
                            rows_v[rb, r, sl] = rows_v[rb, r, sl] * ws
                        return carry
                    lax.fori_loop(0, CHUNK, rm, 0)

                    pltpu.async_copy(rows_v.at[rb], acc.at[sidx_v.at[rb]],
                                     ssem.at[rb], add=True)
                    return gc + 1
                gc = lax.fori_loop(0, nck, cchunk, gc)

                @pl.when(i < SUPS_PER_TILE // 2 - 1)
                def _prefetch():
                    stage(bb, sup + 2)
                return gc

            stage(0, 0)
            stage(1, 1)

            def pair(i, gc):
                gc = super_body(0, 2 * i, i, gc)
                gc = super_body(1, 2 * i + 1, i, gc)
                return gc
            gc = lax.fori_loop(0, SUPS_PER_TILE // 2, pair, gc)

        @pl.when(gc >= 2)
        def _d0():
            pltpu.make_async_copy(out_hbm.at[pl.ds(0, CHUNK)],
                                  rows_v.at[gc & 1], ssem.at[gc & 1]).wait()

        @pl.when(gc >= 1)
        def _d1():
            pltpu.make_async_copy(out_hbm.at[pl.ds(0, CHUNK)],
                                  rows_v.at[(gc - 1) & 1],
                                  ssem.at[(gc - 1) & 1]).wait()
        plsc.subcore_barrier()

        for half in range(2):
            boff = s * 256 + half * CHUNK
            pltpu.sync_copy(bat_hbm.at[pl.ds(boff, CHUNK)], bidx_v)

            def iloop(j, carry):
                sl = pl.ds(j * 16, 16)
                sidx_v[0, sl] = plsc.load_gather(inv_v, [bidx_v[sl]])
                return carry
            lax.fori_loop(0, CHUNK // 16, iloop, 0)
            cpa = pltpu.async_copy(t0_hbm.at[bidx_v], rows_v.at[0], gsem)
            cpb = pltpu.async_copy(t1_hbm.at[bidx_v], rows_v.at[1],
                                   ssem.at[0])
            cpa.wait()
            cpb.wait()

            def addl(r, carry):
                for d in range(4):
                    sl = pl.ds(d * 16, 16)
                    rows_v[0, r, sl] = rows_v[0, r, sl] + rows_v[1, r, sl]
                return carry
            lax.fori_loop(0, CHUNK, addl, 0)
            pltpu.async_copy(acc.at[sidx_v.at[0]], rows_v.at[1],
                             gsem).wait()
            lax.fori_loop(0, CHUNK, addl, 0)
            pltpu.sync_copy(rows_v.at[0], out_hbm.at[pl.ds(boff, CHUNK)])

    @pl.when(c == 0)
    def _userside():
        side(ui_hbm, u0, u1, i1, su_out,
             [(s0, d0, w0, tw0), (s1, d1, w1, tw1)])

    @pl.when(c == 1)
    def _itemside():
        side(ii_hbm, i0, i1, u1, si_out,
             [(d0, s0, w0, tw0), (d1, s1, w1, tw1)])


@functools.partial(
    pl.kernel,
    out_type=jax.ShapeDtypeStruct((4096, ), jnp.float32),
    mesh=_mesh,
    compiler_params=pltpu.CompilerParams(use_tc_tiling_on_sc=False, needs_layout_passes=False),
    scratch_types=[
        pltpu.VMEM((128, D), jnp.float32),
        pltpu.VMEM((128, D), jnp.float32),
        pltpu.VMEM((128,), jnp.float32),
        pltpu.SemaphoreType.DMA,
    ],
)
def _final_kernel(su, si, out_hbm, ur, ir, out_v, sem):
    wid = lax.axis_index("s") * 2 + lax.axis_index("c")
    base = wid * 128
    cpa = pltpu.async_copy(su.at[pl.ds(base, 128)], ur, sem)
    cpb = pltpu.async_copy(si.at[pl.ds(base, 128)], ir, sem)
    cpa.wait()
    cpb.wait()
    lane0 = lax.iota(jnp.int32, 16) == 0

    def row(r, carry):
        p = jnp.zeros((16,), jnp.float32)
        for d in range(4):
            sl = pl.ds(d * 16, 16)
            p = p + ur[r, sl] * ir[r, sl]
        tot = _lane_sum(p) * jnp.float32(1.0 / 9.0)
        plsc.store_scatter(out_v, [jnp.full((16,), r, jnp.int32)],
                           tot, mask=lane0)
        return carry
    lax.fori_loop(0, 128, row, 0)
    pltpu.sync_copy(out_v, out_hbm.at[pl.ds(base, 128)])


def kernel(user_indices, item_indices, edge_index_t0, weights_t0,
           edge_index_t1, weights_t1, user_table, item_table, type_weights):
    epad = E_PAD - E_RAW
    s0 = jnp.pad(edge_index_t0[0], (0, epad))
    d0 = jnp.pad(edge_index_t0[1], (0, epad))
    w0 = jnp.pad(weights_t0, (0, epad))
    s1 = jnp.pad(edge_index_t1[0], (0, epad))
    d1 = jnp.pad(edge_index_t1[1], (0, epad))
    w1 = jnp.pad(weights_t1, (0, epad))
    u0p = jnp.pad(user_table, ((0, NPAD - N_NODES), (0, 0)))
    i0p = jnp.pad(item_table, ((0, NPAD - N_NODES), (0, 0)))
    twp = jnp.pad(type_weights, (0, 14), constant_values=-1e30)

    u1p, i1p = _layer_kernel(u0p, i0p, s0, d0, w0, s1, d1, w1, twp)
    su, si = _tail_kernel(u0p, i0p, u1p, i1p, s0, d0, w0, s1, d1, w1, twp,
                          user_indices, item_indices)
    return _final_kernel(su, si)
